# Initial kernel scaffold; baseline (speedup 1.0000x reference)
import jax
import jax.numpy as jnp
from jax.experimental import pallas as pl


def kernel(input, target):
    raise NotImplementedError("write your pallas kernel here")


# speedup vs baseline: 53.5719x; 53.5719x over previous
import functools

import jax
import jax.numpy as jnp
from jax import lax
from jax.experimental import pallas as pl
from jax.experimental.pallas import tpu as pltpu
from jax.experimental.pallas import tpu_sc as plsc

ROWS = 128
COLS = 100000
NC, NS = 2, 16
NW = NC * NS
RPW = ROWS // NW
CH = 20000
NCHUNK = COLS // CH
LANES = 16
UNROLL = 10
NEG = -3.0e38


def _body(inp_hbm, tgt_hbm, out_hbm, bufa, bufb, tgt_v, out_v, sema, semb):
    wid = lax.axis_index("c") * NS + lax.axis_index("s")
    pltpu.sync_copy(tgt_hbm, tgt_v)

    bufs = (bufa, bufb)
    sems = (sema, semb)

    def slot_src(s):
        i, c = divmod(s, NCHUNK)
        row = wid * RPW + i
        off = row * COLS + c * CH
        return inp_hbm.at[pl.ds(off, CH)]

    def start(s):
        pltpu.make_async_copy(slot_src(s), bufs[s % 2], sems[s % 2]).start()

    def wait(s):
        pltpu.make_async_copy(slot_src(s), bufs[s % 2], sems[s % 2]).wait()

    start(0)

    lane = lax.iota(jnp.int32, LANES)
    loss_acc = jnp.zeros((LANES,), jnp.float32)

    for i in range(RPW):
        row = wid * RPW + i
        y_vec = plsc.load_gather(tgt_v, [jnp.broadcast_to(row, (LANES,))])
        t1 = jnp.full((LANES,), NEG, jnp.float32)
        t2 = jnp.full((LANES,), NEG, jnp.float32)
        t3 = jnp.full((LANES,), NEG, jnp.float32)
        xy = jnp.zeros((LANES,), jnp.float32)

        for c in range(NCHUNK):
            s = i * NCHUNK + c
            if s + 1 < RPW * NCHUNK:
                start(s + 1)
            wait(s)
            buf = bufs[s % 2]

            def body(j, carry):
                a1, a2, a3 = carry
                for k in range(UNROLL):
                    v = buf[pl.ds(j * (LANES * UNROLL) + k * LANES, LANES)]
                    m = jnp.minimum(a1, v)
                    a1 = jnp.maximum(a1, v)
                    m2 = jnp.minimum(a2, m)
                    a2 = jnp.maximum(a2, m)
                    a3 = jnp.maximum(a3, m2)
                return a1, a2, a3

            t1, t2, t3 = lax.fori_loop(
                0, CH // (LANES * UNROLL), body, (t1, t2, t3))

            c0 = c * CH
            in_rng = (y_vec >= c0) & (y_vec < c0 + CH)
            off = jnp.minimum(jnp.maximum(y_vec - c0, 0), CH - 1)
            g = plsc.load_gather(buf, [off])
            xy = jnp.where(in_rng, g, xy)

        m1 = jnp.max(t1)
        m1s = jnp.broadcast_to(m1, (LANES,))
        c1 = (plsc.all_reduce_population_count(t1 == m1s)
              + plsc.all_reduce_population_count(t2 == m1s)
              + plsc.all_reduce_population_count(t3 == m1s))
        u1 = jnp.where(t1 < m1s, t1, NEG)
        u2 = jnp.where(t2 < m1s, t2, NEG)
        u3 = jnp.where(t3 < m1s, t3, NEG)
        n1 = jnp.maximum(jnp.maximum(jnp.max(u1), jnp.max(u2)), jnp.max(u3))
        n1s = jnp.broadcast_to(n1, (LANES,))
        c2 = (plsc.all_reduce_population_count(t1 == n1s)
              + plsc.all_reduce_population_count(t2 == n1s)
              + plsc.all_reduce_population_count(t3 == n1s))
        w1 = jnp.where(u1 < n1s, u1, NEG)
        w2 = jnp.where(u2 < n1s, u2, NEG)
        w3 = jnp.where(u3 < n1s, u3, NEG)
        n2 = jnp.maximum(jnp.maximum(jnp.max(w1), jnp.max(w2)), jnp.max(w3))
        n2s = jnp.broadcast_to(n2, (LANES,))

        m2s = jnp.where(c1 >= 2, m1s, n1s)
        m3s = jnp.where(c1 >= 3, m1s,
                        jnp.where((c1 == 2) | (c2 >= 2), n1s, n2s))

        ind = xy == m1s
        num = -(xy - jnp.where(ind, m2s, m1s))
        den = m1s - m3s + jnp.float32(1e-12)
        loss_acc = loss_acc + num / den

    out_v[...] = jnp.where(lane == 0, loss_acc, jnp.float32(0.0))
    pltpu.sync_copy(out_v, out_hbm.at[pl.ds(wid * LANES, LANES)])


@jax.jit
def _dlr_loss(inp_flat, target):
    out = pl.kernel(
        _body,
        out_type=jax.ShapeDtypeStruct((NW * LANES,), jnp.float32),
        mesh=plsc.VectorSubcoreMesh(
            core_axis_name="c", subcore_axis_name="s",
            num_cores=NC, num_subcores=NS),
        scratch_types=[
            pltpu.VMEM((CH,), jnp.float32),
            pltpu.VMEM((CH,), jnp.float32),
            pltpu.VMEM((ROWS,), jnp.int32),
            pltpu.VMEM((LANES,), jnp.float32),
            pltpu.SemaphoreType.DMA,
            pltpu.SemaphoreType.DMA,
        ],
        compiler_params=pltpu.CompilerParams(needs_layout_passes=False),
    )(inp_flat, target)
    return jnp.sum(out) / jnp.float32(ROWS)


def kernel(input, target):
    return _dlr_loss(input.reshape(-1), target)
